# baseline (speedup 1.0000x reference)
import math

import jax
import jax.numpy as jnp
from jax import lax
from jax.experimental import pallas as pl
from jax.experimental.pallas import tpu as pltpu
from jax.experimental.pallas import tpu_sc as plsc

VOCAB_SIZE = 100000
D_MODEL = 128
BATCH = 4096
SEQ_LEN = 50
SCALE = math.sqrt(D_MODEL)

NC = 2
NS = 16
NW = NC * NS

COLS = BATCH // NW
CHUNK = 64
N_CHUNKS = SEQ_LEN * COLS // CHUNK
NBUF = 10


def _body(x_hbm, w_hbm, out_hbm, idx_v, rows_v, gsem, ssem):
    wid = lax.axis_index("s") * NC + lax.axis_index("c")
    pltpu.sync_copy(x_hbm.at[:, wid], idx_v)

    def gather(c, buf, wait):
        src = w_hbm.at[idx_v.at[c // 2, pl.ds((c % 2) * CHUNK, CHUNK)]]
        dst = rows_v.at[buf]
        if wait:
            pltpu.make_async_copy(src, dst, gsem).wait()
        else:
            pltpu.async_copy(src, dst, gsem)

    def scatter(c, buf, wait):
        src = rows_v.at[buf]
        dst = out_hbm.at[
            pl.ds((c // 2) * BATCH + wid * COLS + (c % 2) * CHUNK, CHUNK)
        ]
        if wait:
            pltpu.make_async_copy(src, dst, ssem).wait()
        else:
            pltpu.async_copy(src, dst, ssem)

    for b in range(NBUF - 1):
        gather(b, b, wait=False)

    @pl.loop(0, N_CHUNKS, step=NBUF)
    def outer(c0):
        for k in range(NBUF):
            c = c0 + k
            prev = (k - 1) % NBUF
            gather(c, k, wait=True)

            @pl.when(c > 0)
            def _():
                scatter(c - 1, prev, wait=True)

            @pl.when(c + NBUF - 1 < N_CHUNKS)
            def _():
                gather(c + NBUF - 1, prev, wait=False)

            @plsc.parallel_loop(0, CHUNK, unroll=4)
            def scale_row(r):
                for j in range(D_MODEL // 16):
                    rows_v[k, r, pl.ds(j * 16, 16)] = (
                        rows_v[k, r, pl.ds(j * 16, 16)] * SCALE
                    )

            scatter(c, k, wait=False)

    scatter(N_CHUNKS - 1, (N_CHUNKS - 1) % NBUF, wait=True)


@jax.jit
def kernel(x, weight):
    xt = x.T.reshape(SEQ_LEN, NW, COLS)
    mesh = plsc.VectorSubcoreMesh(
        core_axis_name="c", subcore_axis_name="s", num_cores=NC, num_subcores=NS
    )
    out = pl.kernel(
        _body,
        out_type=jax.ShapeDtypeStruct((SEQ_LEN * BATCH, D_MODEL), jnp.float32),
        mesh=mesh,
        scratch_types=[
            pltpu.VMEM((SEQ_LEN, COLS), jnp.int32),
            pltpu.VMEM((NBUF, CHUNK, D_MODEL), jnp.float32),
            pltpu.SemaphoreType.DMA,
            pltpu.SemaphoreType.DMA,
        ],
    )(xt, weight)
    return out.reshape(SEQ_LEN, BATCH, D_MODEL).transpose(1, 0, 2)


# speedup vs baseline: 1.0152x; 1.0152x over previous
import math

import jax
import jax.numpy as jnp
from jax import lax
from jax.experimental import pallas as pl
from jax.experimental.pallas import tpu as pltpu
from jax.experimental.pallas import tpu_sc as plsc

VOCAB_SIZE = 100000
D_MODEL = 128
BATCH = 4096
SEQ_LEN = 50
SCALE = math.sqrt(D_MODEL)

NC = 2
NS = 16
NW = NC * NS

CHUNK = BATCH // NW
NBUF = 5


def _body(x_hbm, w_hbm, out_hbm, idx_v, rows_v, gsem, ssem):
    wid = lax.axis_index("s") * NC + lax.axis_index("c")
    pltpu.sync_copy(x_hbm.at[:, wid], idx_v)

    def gather(l, buf, wait):
        src = w_hbm.at[idx_v.at[l]]
        dst = rows_v.at[buf]
        if wait:
            pltpu.make_async_copy(src, dst, gsem).wait()
        else:
            pltpu.async_copy(src, dst, gsem)

    def scatter(l, buf, wait):
        src = rows_v.at[buf]
        dst = out_hbm.at[pl.ds(l * BATCH + wid * CHUNK, CHUNK)]
        if wait:
            pltpu.make_async_copy(src, dst, ssem).wait()
        else:
            pltpu.async_copy(src, dst, ssem)

    for b in range(NBUF - 1):
        gather(b, b, wait=False)

    @pl.loop(0, SEQ_LEN, step=NBUF)
    def outer(l0):
        for k in range(NBUF):
            l = l0 + k
            prev = (k - 1) % NBUF
            gather(l, k, wait=True)

            @pl.when(l > 0)
            def _():
                scatter(l - 1, prev, wait=True)

            @pl.when(l + NBUF - 1 < SEQ_LEN)
            def _():
                gather(l + NBUF - 1, prev, wait=False)

            @plsc.parallel_loop(0, CHUNK, unroll=4)
            def scale_row(r):
                for j in range(D_MODEL // 16):
                    rows_v[k, r, pl.ds(j * 16, 16)] = (
                        rows_v[k, r, pl.ds(j * 16, 16)] * SCALE
                    )

            scatter(l, k, wait=False)

    scatter(SEQ_LEN - 1, (SEQ_LEN - 1) % NBUF, wait=True)


@jax.jit
def kernel(x, weight):
    xt = x.T.reshape(SEQ_LEN, NW, CHUNK)
    mesh = plsc.VectorSubcoreMesh(
        core_axis_name="c", subcore_axis_name="s", num_cores=NC, num_subcores=NS
    )
    out = pl.kernel(
        _body,
        out_type=jax.ShapeDtypeStruct((SEQ_LEN * BATCH, D_MODEL), jnp.float32),
        mesh=mesh,
        scratch_types=[
            pltpu.VMEM((SEQ_LEN, CHUNK), jnp.int32),
            pltpu.VMEM((NBUF, CHUNK, D_MODEL), jnp.float32),
            pltpu.SemaphoreType.DMA,
            pltpu.SemaphoreType.DMA,
        ],
    )(xt, weight)
    return out.reshape(SEQ_LEN, BATCH, D_MODEL).transpose(1, 0, 2)
